# Initial kernel scaffold; baseline (speedup 1.0000x reference)
import jax
import jax.numpy as jnp
from jax.experimental import pallas as pl


def kernel(boxes, scores):
    raise NotImplementedError("write your pallas kernel here")


# speedup vs baseline: 67.4081x; 67.4081x over previous
import functools

import jax
import jax.numpy as jnp
from jax import lax
from jax.experimental import pallas as pl
from jax.experimental.pallas import tpu as pltpu

_N_BOXES = 20000
_PRE_TOPK = 2000
_POST_TOPK = 1000
_NMS_THRESH = 0.7
_NPAD = 32768
_ROWS = 256
_LANES = 128
_TOP_ROWS = 16
_NEG_INF = float("-inf")


def _bitonic_stage(arrs, ks_pos, ki_pos, d, blk, ri, ci, descending):
  if d < _LANES:
    def partner(a):
      lo = pltpu.roll(a, _LANES - d, axis=1)
      hi = pltpu.roll(a, d, axis=1)
      return jnp.where((ci & d) != 0, hi, lo)
  else:
    m = d // _LANES
    rows = arrs[0].shape[0]
    g = rows // (2 * m)
    def partner(a):
      a4 = a.reshape(g, 2, m, _LANES)
      a4 = jnp.concatenate([a4[:, 1:2], a4[:, 0:1]], axis=1)
      return a4.reshape(rows, _LANES)

  parts = [partner(a) for a in arrs]
  p = ri * _LANES + ci
  key_s, sq = arrs[ks_pos], parts[ks_pos]
  if descending:
    key_i, iq = arrs[ki_pos], parts[ki_pos]
    mine_first = (key_s > sq) | ((key_s == sq) & (key_i < iq))
  else:
    mine_first = key_s < sq
  am_high = (p & d) != 0
  blk_flip = (p & blk) != 0
  keep_mine = (mine_first != am_high) != blk_flip
  return [jnp.where(keep_mine, a, q) for a, q in zip(arrs, parts)]


def _bitonic_sort(arrs, ks_pos, ki_pos, n, ri, ci, descending):
  blk = 2
  while blk <= n:
    d = blk // 2
    while d >= 1:
      arrs = _bitonic_stage(arrs, ks_pos, ki_pos, d, blk, ri, ci,
                            descending)
      d //= 2
    blk *= 2
  return arrs


def _transpose(x, eye):
  del eye
  return jnp.transpose(x)


def _nms_kernel(s_ref, x1_ref, y1_ref, x2_ref, y2_ref,
                os_ref, ox1_ref, oy1_ref, ox2_ref, oy2_ref, m_ref):
  ri = lax.broadcasted_iota(jnp.int32, (_ROWS, _LANES), 0)
  ci = lax.broadcasted_iota(jnp.int32, (_ROWS, _LANES), 1)

  s = s_ref[...]
  idx = ri * _LANES + ci
  arrs = [s, idx, x1_ref[...], y1_ref[...], x2_ref[...], y2_ref[...]]
  arrs = _bitonic_sort(arrs, 0, 1, _NPAD, ri, ci, descending=True)
  s, _, x1, y1, x2, y2 = arrs

  st = s[:_TOP_ROWS]
  x1t = x1[:_TOP_ROWS]
  y1t = y1[:_TOP_ROWS]
  x2t = x2[:_TOP_ROWS]
  y2t = y2[:_TOP_ROWS]
  ri16 = ri[:_TOP_ROWS]
  ci16 = ci[:_TOP_ROWS]
  rank = ri16 * _LANES + ci16

  w = x2t - x1t
  h = y2t - y1t
  valid = (rank < _PRE_TOPK) & (w >= 0.0) & (h >= 0.0)
  validf = valid.astype(jnp.float32)
  area = w * h

  eye = (lax.broadcasted_iota(jnp.int32, (_LANES, _LANES), 0) ==
         lax.broadcasted_iota(jnp.int32, (_LANES, _LANES), 1)
         ).astype(jnp.float32)
  tx1 = _transpose(x1t, eye)
  ty1 = _transpose(y1t, eye)
  tx2 = _transpose(x2t, eye)
  ty2 = _transpose(y2t, eye)
  tarea = _transpose(area, eye)

  io_r = lax.broadcasted_iota(jnp.int32, (_LANES, _LANES), 0)
  io_c = lax.broadcasted_iota(jnp.int32, (_LANES, _LANES), 1)

  tile_of = {}
  t = 0
  for b in range(_TOP_ROWS):
    for a in range(b + 1):
      tile_of[(a, b)] = t
      t += 1
  for b in range(_TOP_ROWS):
    xb1 = x1t[b:b + 1, :]
    yb1 = y1t[b:b + 1, :]
    xb2 = x2t[b:b + 1, :]
    yb2 = y2t[b:b + 1, :]
    ab = area[b:b + 1, :]
    for a in range(b + 1):
      xa1 = tx1[:, a:a + 1]
      ya1 = ty1[:, a:a + 1]
      xa2 = tx2[:, a:a + 1]
      ya2 = ty2[:, a:a + 1]
      aa = tarea[:, a:a + 1]
      iw = jnp.clip(jnp.minimum(xa2, xb2) - jnp.maximum(xa1, xb1), 0.0)
      ih = jnp.clip(jnp.minimum(ya2, yb2) - jnp.maximum(ya1, yb1), 0.0)
      inter = iw * ih
      union = aa + ab - inter
      over = inter / jnp.maximum(union, 1e-9) > _NMS_THRESH
      if a == b:
        over = over & (io_r < io_c)
      ofs = tile_of[(a, b)] * _LANES
      m_ref[ofs:ofs + _LANES, :] = over.astype(jnp.float32)

  def col(row_vec):
    return jnp.transpose(row_vec)

  def sweep(carry):
    k, _ = carry
    cols_old = _transpose(k, eye)
    new_rows = []
    new_cols = []
    for b in range(_TOP_ROWS):
      acc = jnp.zeros((1, _LANES), jnp.float32)
      for a in range(b):
        ofs = tile_of[(a, b)] * _LANES
        acc = acc + jnp.sum(m_ref[ofs:ofs + _LANES, :] * new_cols[a],
                            axis=0, keepdims=True)
      ofs = tile_of[(b, b)] * _LANES
      acc = acc + jnp.sum(m_ref[ofs:ofs + _LANES, :] * cols_old[:, b:b + 1],
                          axis=0, keepdims=True)
      row = validf[b:b + 1, :] * (acc <= 0.0).astype(jnp.float32)
      new_rows.append(row)
      new_cols.append(col(row))
    knew = jnp.concatenate(new_rows, axis=0)
    done = jnp.sum(jnp.abs(knew - k)) == 0.0
    return knew, done

  k0 = validf
  kfin, _ = lax.while_loop(lambda c: jnp.logical_not(c[1]), sweep,
                           (k0, jnp.asarray(False)))

  kept = kfin > 0.0
  out_s = jnp.where(kept, st, _NEG_INF)
  key = rank + jnp.where(kept, 0, 4096)
  arrs2 = [key, out_s, x1t, y1t, x2t, y2t]
  arrs2 = _bitonic_sort(arrs2, 0, None, _TOP_ROWS * _LANES, ri16, ci16,
                        descending=False)
  _, fs, fx1, fy1, fx2, fy2 = arrs2

  os_ref[...] = fs[:8]
  ox1_ref[...] = fx1[:8]
  oy1_ref[...] = fy1[:8]
  ox2_ref[...] = fx2[:8]
  oy2_ref[...] = fy2[:8]


@jax.jit
def kernel(boxes, scores):
  spad = jnp.full((_NPAD,), _NEG_INF, jnp.float32).at[:_N_BOXES].set(scores)
  coords = []
  for c in range(4):
    coords.append(
        jnp.zeros((_NPAD,), jnp.float32).at[:_N_BOXES].set(boxes[:, c])
        .reshape(_ROWS, _LANES))
  s2d = spad.reshape(_ROWS, _LANES)

  out_shapes = [jax.ShapeDtypeStruct((8, _LANES), jnp.float32)] * 5
  outs = pl.pallas_call(
      _nms_kernel,
      out_shape=out_shapes,
      scratch_shapes=[pltpu.VMEM((136 * _LANES, _LANES), jnp.float32)],
  )(s2d, *coords)
  fs, fx1, fy1, fx2, fy2 = outs
  out_s = fs.reshape(8 * _LANES)[:_POST_TOPK]
  out_b = jnp.stack(
      [fx1.reshape(8 * _LANES)[:_POST_TOPK],
       fy1.reshape(8 * _LANES)[:_POST_TOPK],
       fx2.reshape(8 * _LANES)[:_POST_TOPK],
       fy2.reshape(8 * _LANES)[:_POST_TOPK]], axis=1)
  return out_b, out_s
